# baseline (speedup 1.0000x reference)
import functools

import jax
import jax.numpy as jnp
from jax import lax
from jax.experimental import pallas as pl
from jax.experimental.pallas import tpu as pltpu
from jax.experimental.pallas import tpu_sc as plsc

_NEG_POS_RATIO = 3
_ALPHA = 1.0


def _phase1_body(tbl_ref, lg_ref, anT_ref, poT_ref, ml_ref,
                 cls_ref, bits_ref, reg_ref):
    first = (pl.program_id(0) == 0) & (pl.program_id(1) == 0)
    RB = ml_ref.shape[1]
    L = ml_ref.shape[2]
    C = lg_ref.shape[3]
    G = tbl_ref.shape[2]

    m = ml_ref[0]
    fg = m >= 0
    safe = jnp.maximum(m, 0)

    zero = jnp.zeros((RB, L), jnp.float32)
    gx0 = zero
    gy0 = zero
    gx1 = zero
    gy1 = zero
    lab = zero
    for g in range(G):
        sel = safe == g
        gx0 = jnp.where(sel, tbl_ref[0, 0, g], gx0)
        gy0 = jnp.where(sel, tbl_ref[0, 1, g], gy0)
        gx1 = jnp.where(sel, tbl_ref[0, 2, g], gx1)
        gy1 = jnp.where(sel, tbl_ref[0, 3, g], gy1)
        lab = jnp.where(sel, tbl_ref[0, 4, g], lab)

    an = anT_ref[0]
    po = poT_ref[0]
    ax0 = an[0]
    ay0 = an[1]
    ax1 = an[2]
    ay1 = an[3]
    aw = ax1 - ax0
    ah = ay1 - ay0
    t0 = ((gx0 + gx1) - (ax0 + ax1)) * 0.5 / aw
    t1 = ((gy0 + gy1) - (ay0 + ay1)) * 0.5 / ah
    t2 = jnp.log((gx1 - gx0) / aw)
    t3 = jnp.log((gy1 - gy0) / ah)
    sl1 = zero
    for j, tj in enumerate((t0, t1, t2, t3)):
        d = jnp.abs(po[j] - tj)
        sl1 = sl1 + jnp.where(d < 1.0, 0.5 * d * d, d - 0.5)
    regp = jnp.sum(jnp.where(fg, sl1, 0.0))

    lg3 = lg_ref[0]
    mx = jnp.max(lg3, axis=2)
    e = jnp.exp(lg3 - mx[:, :, None])
    s = jnp.sum(e, axis=2)
    lse = mx + jnp.log(s)

    acls = jnp.where(fg, lab.astype(jnp.int32), 0)
    cio = jax.lax.broadcasted_iota(jnp.int32, (RB, L, C), 2)
    picked = jnp.sum(jnp.where(cio == acls[:, :, None], lg3, 0.0), axis=2)
    clsv = lse - picked
    cls_ref[0] = clsv
    bits_ref[0] = jax.lax.bitcast_convert_type(jnp.maximum(clsv, 0.0),
                                               jnp.int32)

    @pl.when(first)
    def _():
        reg_ref[...] = jnp.zeros((1, 1), jnp.float32)
    reg_ref[...] += regp.reshape(1, 1)


def _mine_body(cls_hbm, bits_hbm, m_hbm, out_hbm, vals_v, m_v, bits_v,
               outv_v):
    wid = lax.axis_index("s") * 2 + lax.axis_index("c")
    A = vals_v.shape[0]
    NV = A // 16

    pltpu.sync_copy(cls_hbm.at[pl.ds(wid * A, A)], vals_v)
    pltpu.sync_copy(bits_hbm.at[pl.ds(wid * A, A)], bits_v)
    pltpu.sync_copy(m_hbm.at[pl.ds(wid * A, A)], m_v)

    zi16 = jnp.zeros((16,), jnp.int32)
    zf16 = jnp.zeros((16,), jnp.float32)
    iota = lax.broadcasted_iota(jnp.int32, (16,), 0)

    def ssum_i(vec):
        s = vec[0]
        for l in range(1, 16):
            s = s + vec[l]
        return s

    def ssum_f(vec):
        s = vec[0]
        for l in range(1, 16):
            s = s + vec[l]
        return s

    def smax_f(vec):
        s = vec[0]
        for l in range(1, 16):
            s = jnp.maximum(s, vec[l])
        return s

    def prep(i, acc):
        mm = m_v[pl.ds(i * 16, 16)]
        posm = mm >= 0
        b = jnp.where(posm, -1, bits_v[pl.ds(i * 16, 16)])
        bits_v[pl.ds(i * 16, 16)] = b
        return acc + jnp.where(posm, 1, 0)

    num_pos = ssum_i(lax.fori_loop(0, NV, prep, zi16))
    n_neg = A - num_pos
    k0 = jnp.minimum(3 * num_pos, n_neg)

    t = jnp.int32(0)
    for r in range(31):
        cand = t | jnp.int32(1 << (30 - r))

        def cbody(i, acc, cand=cand):
            for u in range(4):
                b = bits_v[pl.ds((i * 4 + u) * 16, 16)]
                acc = acc + jnp.where(b >= cand, 1, 0)
            return acc

        cnt = ssum_i(lax.fori_loop(0, NV // 4, cbody, zi16))
        t = jnp.where(cnt >= k0, cand, t)

    def fin(i, carry):
        psum, gsum, gcnt, tmax = carry
        for u in range(2):
            v = jnp.maximum(vals_v[pl.ds((i * 2 + u) * 16, 16)], 0.0)
            b = bits_v[pl.ds((i * 2 + u) * 16, 16)]
            psum = psum + jnp.where(b < 0, v, 0.0)
            gtm = b > t
            gsum = gsum + jnp.where(gtm, v, 0.0)
            gcnt = gcnt + jnp.where(gtm, 1, 0)
            tmax = jnp.maximum(tmax, jnp.where(b == t, v, 0.0))
        return (psum, gsum, gcnt, tmax)

    psum, gsum, gcnt, tmax = lax.fori_loop(0, NV // 2, fin,
                                           (zf16, zf16, zi16, zf16))
    pos_sum = ssum_f(psum)
    sum_gt = ssum_f(gsum)
    cnt_gt = ssum_i(gcnt)
    thr = smax_f(tmax)
    neg_sum = jnp.where(k0 > 0,
                        sum_gt + (k0 - cnt_gt).astype(jnp.float32) * thr,
                        0.0)
    outv_v[...] = jnp.where(
        iota == 0, pos_sum + neg_sum,
        jnp.where(iota == 1, num_pos.astype(jnp.float32), 0.0))
    pltpu.sync_copy(outv_v, out_hbm.at[pl.ds(wid * 16, 16)])


def _combine_body(rr_ref, reg_ref, out_ref):
    rr = rr_ref[...]
    s0 = jnp.sum(rr[:, 0:1])
    npos = jnp.sum(rr[:, 1:2])
    out_ref[...] = (_ALPHA * reg_ref[...]
                    + (s0 / jnp.maximum(npos, 1.0)).reshape(1, 1))


def _phase2_body(cls_ref, m_ref, reg_ref, out_ref):
    cls = cls_ref[...]
    m = m_ref[...]
    B = cls.shape[0]
    A = cls.shape[1]
    pos = m >= 0
    neg = jnp.logical_not(pos)
    posi = pos.astype(jnp.int32)
    num_pos = jnp.sum(posi, axis=1, keepdims=True)
    k = jnp.minimum(_NEG_POS_RATIO * num_pos, A - num_pos)

    clsc = jnp.maximum(cls, 0.0)
    bits = jax.lax.bitcast_convert_type(clsc, jnp.int32)
    bits = jnp.where(neg, bits, -1)

    def body(i, t):
        cand = t | jax.lax.shift_right_logical(jnp.int32(2 ** 30), i)
        cnt = jnp.sum((bits >= cand).astype(jnp.int32), axis=1,
                      keepdims=True)
        return jnp.where(cnt >= k, cand, t)

    t = jax.lax.fori_loop(0, 31, body, jnp.zeros((B, 1), jnp.int32))

    thr = jax.lax.bitcast_convert_type(t, jnp.float32)
    gt = neg & (bits > t)
    cnt_gt = jnp.sum(gt.astype(jnp.int32), axis=1, keepdims=True)
    sum_gt = jnp.sum(jnp.where(gt, clsc, 0.0), axis=1, keepdims=True)
    neg_sum = jnp.where(k > 0,
                        sum_gt + (k - cnt_gt).astype(jnp.float32) * thr,
                        0.0)
    pos_sum = jnp.sum(jnp.where(pos, clsc, 0.0), axis=1, keepdims=True)
    npt = jnp.maximum(1, jnp.sum(num_pos)).astype(jnp.float32)
    final_cls = (jnp.sum(pos_sum) + jnp.sum(neg_sum)) / npt
    out_ref[...] = _ALPHA * reg_ref[...] + final_cls.reshape(1, 1)


@jax.jit
def kernel(targets_bbox, targets_labels, pred_offsets, pred_cls_logits,
           anchors, matches):
    B, A, C = pred_cls_logits.shape
    G = targets_labels.shape[1]
    L = 128
    BLK = 8192
    RB = BLK // L

    tbl = jnp.concatenate(
        [jnp.transpose(targets_bbox, (0, 2, 1)),
         targets_labels.astype(jnp.float32)[:, None, :]], axis=1)
    lg4 = pred_cls_logits.reshape(B, A // L, L, C)
    anT = jnp.transpose(anchors, (0, 2, 1)).reshape(B, 4, A // L, L)
    poT = jnp.transpose(pred_offsets, (0, 2, 1)).reshape(B, 4, A // L, L)
    ml = matches.astype(jnp.int32).reshape(B, A // L, L)

    cls, clsbits, reg = pl.pallas_call(
        _phase1_body,
        grid=(B, A // BLK),
        in_specs=[
            pl.BlockSpec((1, 5, G), lambda b, j: (b, 0, 0),
                         memory_space=pltpu.SMEM),
            pl.BlockSpec((1, RB, L, C), lambda b, j: (b, j, 0, 0)),
            pl.BlockSpec((1, 4, RB, L), lambda b, j: (b, 0, j, 0)),
            pl.BlockSpec((1, 4, RB, L), lambda b, j: (b, 0, j, 0)),
            pl.BlockSpec((1, RB, L), lambda b, j: (b, j, 0)),
        ],
        out_specs=[
            pl.BlockSpec((1, RB, L), lambda b, j: (b, j, 0)),
            pl.BlockSpec((1, RB, L), lambda b, j: (b, j, 0)),
            pl.BlockSpec((1, 1), lambda b, j: (0, 0)),
        ],
        out_shape=[
            jax.ShapeDtypeStruct((B, A // L, L), jnp.float32),
            jax.ShapeDtypeStruct((B, A // L, L), jnp.int32),
            jax.ShapeDtypeStruct((1, 1), jnp.float32),
        ],
    )(tbl, lg4, anT, poT, ml)

    mine = functools.partial(
        pl.kernel,
        mesh=plsc.VectorSubcoreMesh(core_axis_name="c",
                                    subcore_axis_name="s"),
        out_type=jax.ShapeDtypeStruct((B * 16,), jnp.float32),
        scratch_types=[
            pltpu.VMEM((A,), jnp.float32),
            pltpu.VMEM((A,), jnp.int32),
            pltpu.VMEM((A,), jnp.int32),
            pltpu.VMEM((16,), jnp.float32),
        ],
    )(_mine_body)
    rowres = mine(cls.reshape(B * A), clsbits.reshape(B * A),
                  matches.astype(jnp.int32).reshape(B * A))

    out = pl.pallas_call(
        _combine_body,
        out_shape=jax.ShapeDtypeStruct((1, 1), jnp.float32),
    )(rowres.reshape(B, 16), reg)
    return out.reshape(())


# speedup vs baseline: 1.7704x; 1.7704x over previous
import functools

import jax
import jax.numpy as jnp
from jax import lax
from jax.experimental import pallas as pl
from jax.experimental.pallas import tpu as pltpu
from jax.experimental.pallas import tpu_sc as plsc

_NEG_POS_RATIO = 3
_ALPHA = 1.0


def _phase1_body(tbl_ref, lg_ref, anT_ref, poT_ref, ml_ref,
                 cls_ref, bits_ref, reg_ref):
    first = (pl.program_id(0) == 0) & (pl.program_id(1) == 0)
    RB = ml_ref.shape[1]
    L = ml_ref.shape[2]
    C = lg_ref.shape[3]
    G = tbl_ref.shape[2]

    m = ml_ref[0]
    fg = m >= 0
    safe = jnp.maximum(m, 0)

    zero = jnp.zeros((RB, L), jnp.float32)
    gx0 = zero
    gy0 = zero
    gx1 = zero
    gy1 = zero
    lab = zero
    for g in range(G):
        sel = safe == g
        gx0 = jnp.where(sel, tbl_ref[0, 0, g], gx0)
        gy0 = jnp.where(sel, tbl_ref[0, 1, g], gy0)
        gx1 = jnp.where(sel, tbl_ref[0, 2, g], gx1)
        gy1 = jnp.where(sel, tbl_ref[0, 3, g], gy1)
        lab = jnp.where(sel, tbl_ref[0, 4, g], lab)

    an = anT_ref[0]
    po = poT_ref[0]
    ax0 = an[0]
    ay0 = an[1]
    ax1 = an[2]
    ay1 = an[3]
    aw = ax1 - ax0
    ah = ay1 - ay0
    t0 = ((gx0 + gx1) - (ax0 + ax1)) * 0.5 / aw
    t1 = ((gy0 + gy1) - (ay0 + ay1)) * 0.5 / ah
    t2 = jnp.log((gx1 - gx0) / aw)
    t3 = jnp.log((gy1 - gy0) / ah)
    sl1 = zero
    for j, tj in enumerate((t0, t1, t2, t3)):
        d = jnp.abs(po[j] - tj)
        sl1 = sl1 + jnp.where(d < 1.0, 0.5 * d * d, d - 0.5)
    regp = jnp.sum(jnp.where(fg, sl1, 0.0))

    lg3 = lg_ref[0]
    acls = jnp.where(fg, lab.astype(jnp.int32), 0)
    cio = jax.lax.broadcasted_iota(jnp.int32, (C, L), 0)
    for i in range(RB):
        tilT = jnp.swapaxes(lg3[i], 0, 1)
        mxr = jnp.max(tilT, axis=0)
        er = jnp.exp(tilT - mxr[None, :])
        sr = jnp.sum(er, axis=0)
        lser = mxr + jnp.log(sr)
        ar = acls[i]
        pickr = jnp.sum(jnp.where(cio == ar[None, :], tilT, 0.0), axis=0)
        clsr = lser - pickr
        cls_ref[0, i] = clsr
        bits_ref[0, i] = jax.lax.bitcast_convert_type(
            jnp.maximum(clsr, 0.0), jnp.int32)

    @pl.when(first)
    def _():
        reg_ref[...] = jnp.zeros((1, 1), jnp.float32)
    reg_ref[...] += regp.reshape(1, 1)


def _mine_body(cls_hbm, bits_hbm, m_hbm, out_hbm, vals_v, m_v, bits_v,
               outv_v):
    wid = lax.axis_index("s") * 2 + lax.axis_index("c")
    A = vals_v.shape[0]
    NV = A // 16

    pltpu.sync_copy(cls_hbm.at[pl.ds(wid * A, A)], vals_v)
    pltpu.sync_copy(bits_hbm.at[pl.ds(wid * A, A)], bits_v)
    pltpu.sync_copy(m_hbm.at[pl.ds(wid * A, A)], m_v)

    zi16 = jnp.zeros((16,), jnp.int32)
    zf16 = jnp.zeros((16,), jnp.float32)
    iota = lax.broadcasted_iota(jnp.int32, (16,), 0)

    def ssum_i(vec):
        s = vec[0]
        for l in range(1, 16):
            s = s + vec[l]
        return s

    def ssum_f(vec):
        s = vec[0]
        for l in range(1, 16):
            s = s + vec[l]
        return s

    def smax_f(vec):
        s = vec[0]
        for l in range(1, 16):
            s = jnp.maximum(s, vec[l])
        return s

    def prep(i, acc):
        mm = m_v[pl.ds(i * 16, 16)]
        posm = mm >= 0
        b = jnp.where(posm, -1, bits_v[pl.ds(i * 16, 16)])
        bits_v[pl.ds(i * 16, 16)] = b
        return acc + jnp.where(posm, 1, 0)

    num_pos = ssum_i(lax.fori_loop(0, NV, prep, zi16))
    n_neg = A - num_pos
    k0 = jnp.minimum(3 * num_pos, n_neg)

    t = jnp.int32(0)
    for r in range(31):
        cand = t | jnp.int32(1 << (30 - r))

        def cbody(i, acc, cand=cand):
            for u in range(4):
                b = bits_v[pl.ds((i * 4 + u) * 16, 16)]
                acc = acc + jnp.where(b >= cand, 1, 0)
            return acc

        cnt = ssum_i(lax.fori_loop(0, NV // 4, cbody, zi16))
        t = jnp.where(cnt >= k0, cand, t)

    def fin(i, carry):
        psum, gsum, gcnt, tmax = carry
        for u in range(2):
            v = jnp.maximum(vals_v[pl.ds((i * 2 + u) * 16, 16)], 0.0)
            b = bits_v[pl.ds((i * 2 + u) * 16, 16)]
            psum = psum + jnp.where(b < 0, v, 0.0)
            gtm = b > t
            gsum = gsum + jnp.where(gtm, v, 0.0)
            gcnt = gcnt + jnp.where(gtm, 1, 0)
            tmax = jnp.maximum(tmax, jnp.where(b == t, v, 0.0))
        return (psum, gsum, gcnt, tmax)

    psum, gsum, gcnt, tmax = lax.fori_loop(0, NV // 2, fin,
                                           (zf16, zf16, zi16, zf16))
    pos_sum = ssum_f(psum)
    sum_gt = ssum_f(gsum)
    cnt_gt = ssum_i(gcnt)
    thr = smax_f(tmax)
    neg_sum = jnp.where(k0 > 0,
                        sum_gt + (k0 - cnt_gt).astype(jnp.float32) * thr,
                        0.0)
    outv_v[...] = jnp.where(
        iota == 0, pos_sum + neg_sum,
        jnp.where(iota == 1, num_pos.astype(jnp.float32), 0.0))
    pltpu.sync_copy(outv_v, out_hbm.at[pl.ds(wid * 16, 16)])


def _combine_body(rr_ref, reg_ref, out_ref):
    rr = rr_ref[...]
    s0 = jnp.sum(rr[:, 0:1])
    npos = jnp.sum(rr[:, 1:2])
    out_ref[...] = (_ALPHA * reg_ref[...]
                    + (s0 / jnp.maximum(npos, 1.0)).reshape(1, 1))


def _phase2_body(cls_ref, m_ref, reg_ref, out_ref):
    cls = cls_ref[...]
    m = m_ref[...]
    B = cls.shape[0]
    A = cls.shape[1]
    pos = m >= 0
    neg = jnp.logical_not(pos)
    posi = pos.astype(jnp.int32)
    num_pos = jnp.sum(posi, axis=1, keepdims=True)
    k = jnp.minimum(_NEG_POS_RATIO * num_pos, A - num_pos)

    clsc = jnp.maximum(cls, 0.0)
    bits = jax.lax.bitcast_convert_type(clsc, jnp.int32)
    bits = jnp.where(neg, bits, -1)

    def body(i, t):
        cand = t | jax.lax.shift_right_logical(jnp.int32(2 ** 30), i)
        cnt = jnp.sum((bits >= cand).astype(jnp.int32), axis=1,
                      keepdims=True)
        return jnp.where(cnt >= k, cand, t)

    t = jax.lax.fori_loop(0, 31, body, jnp.zeros((B, 1), jnp.int32))

    thr = jax.lax.bitcast_convert_type(t, jnp.float32)
    gt = neg & (bits > t)
    cnt_gt = jnp.sum(gt.astype(jnp.int32), axis=1, keepdims=True)
    sum_gt = jnp.sum(jnp.where(gt, clsc, 0.0), axis=1, keepdims=True)
    neg_sum = jnp.where(k > 0,
                        sum_gt + (k - cnt_gt).astype(jnp.float32) * thr,
                        0.0)
    pos_sum = jnp.sum(jnp.where(pos, clsc, 0.0), axis=1, keepdims=True)
    npt = jnp.maximum(1, jnp.sum(num_pos)).astype(jnp.float32)
    final_cls = (jnp.sum(pos_sum) + jnp.sum(neg_sum)) / npt
    out_ref[...] = _ALPHA * reg_ref[...] + final_cls.reshape(1, 1)


@jax.jit
def kernel(targets_bbox, targets_labels, pred_offsets, pred_cls_logits,
           anchors, matches):
    B, A, C = pred_cls_logits.shape
    G = targets_labels.shape[1]
    L = 128
    BLK = 8192
    RB = BLK // L

    tbl = jnp.concatenate(
        [jnp.transpose(targets_bbox, (0, 2, 1)),
         targets_labels.astype(jnp.float32)[:, None, :]], axis=1)
    lg4 = pred_cls_logits.reshape(B, A // L, L, C)
    anT = jnp.transpose(anchors, (0, 2, 1)).reshape(B, 4, A // L, L)
    poT = jnp.transpose(pred_offsets, (0, 2, 1)).reshape(B, 4, A // L, L)
    ml = matches.astype(jnp.int32).reshape(B, A // L, L)

    cls, clsbits, reg = pl.pallas_call(
        _phase1_body,
        grid=(B, A // BLK),
        in_specs=[
            pl.BlockSpec((1, 5, G), lambda b, j: (b, 0, 0),
                         memory_space=pltpu.SMEM),
            pl.BlockSpec((1, RB, L, C), lambda b, j: (b, j, 0, 0)),
            pl.BlockSpec((1, 4, RB, L), lambda b, j: (b, 0, j, 0)),
            pl.BlockSpec((1, 4, RB, L), lambda b, j: (b, 0, j, 0)),
            pl.BlockSpec((1, RB, L), lambda b, j: (b, j, 0)),
        ],
        out_specs=[
            pl.BlockSpec((1, RB, L), lambda b, j: (b, j, 0)),
            pl.BlockSpec((1, RB, L), lambda b, j: (b, j, 0)),
            pl.BlockSpec((1, 1), lambda b, j: (0, 0)),
        ],
        out_shape=[
            jax.ShapeDtypeStruct((B, A // L, L), jnp.float32),
            jax.ShapeDtypeStruct((B, A // L, L), jnp.int32),
            jax.ShapeDtypeStruct((1, 1), jnp.float32),
        ],
    )(tbl, lg4, anT, poT, ml)

    mine = functools.partial(
        pl.kernel,
        mesh=plsc.VectorSubcoreMesh(core_axis_name="c",
                                    subcore_axis_name="s"),
        out_type=jax.ShapeDtypeStruct((B * 16,), jnp.float32),
        scratch_types=[
            pltpu.VMEM((A,), jnp.float32),
            pltpu.VMEM((A,), jnp.int32),
            pltpu.VMEM((A,), jnp.int32),
            pltpu.VMEM((16,), jnp.float32),
        ],
    )(_mine_body)
    rowres = mine(cls.reshape(B * A), clsbits.reshape(B * A),
                  matches.astype(jnp.int32).reshape(B * A))

    out = pl.pallas_call(
        _combine_body,
        out_shape=jax.ShapeDtypeStruct((1, 1), jnp.float32),
    )(rowres.reshape(B, 16), reg)
    return out.reshape(())


# speedup vs baseline: 1.7711x; 1.0004x over previous
import functools

import jax
import jax.numpy as jnp
from jax import lax
from jax.experimental import pallas as pl
from jax.experimental.pallas import tpu as pltpu
from jax.experimental.pallas import tpu_sc as plsc

_NEG_POS_RATIO = 3
_ALPHA = 1.0


def _phase1_body(tbl_ref, lg_ref, anT_ref, poT_ref, ml_ref,
                 cls_ref, bits_ref, reg_ref):
    first = (pl.program_id(0) == 0) & (pl.program_id(1) == 0)
    RB = ml_ref.shape[1]
    L = ml_ref.shape[2]
    C = lg_ref.shape[3]
    G = tbl_ref.shape[2]

    m = ml_ref[0]
    fg = m >= 0
    safe = jnp.maximum(m, 0)

    zero = jnp.zeros((RB, L), jnp.float32)
    gx0 = zero
    gy0 = zero
    gx1 = zero
    gy1 = zero
    lab = zero
    for g in range(G):
        sel = safe == g
        gx0 = jnp.where(sel, tbl_ref[0, 0, g], gx0)
        gy0 = jnp.where(sel, tbl_ref[0, 1, g], gy0)
        gx1 = jnp.where(sel, tbl_ref[0, 2, g], gx1)
        gy1 = jnp.where(sel, tbl_ref[0, 3, g], gy1)
        lab = jnp.where(sel, tbl_ref[0, 4, g], lab)

    an = anT_ref[0]
    po = poT_ref[0]
    ax0 = an[0]
    ay0 = an[1]
    ax1 = an[2]
    ay1 = an[3]
    aw = ax1 - ax0
    ah = ay1 - ay0
    t0 = ((gx0 + gx1) - (ax0 + ax1)) * 0.5 / aw
    t1 = ((gy0 + gy1) - (ay0 + ay1)) * 0.5 / ah
    t2 = jnp.log((gx1 - gx0) / aw)
    t3 = jnp.log((gy1 - gy0) / ah)
    sl1 = zero
    for j, tj in enumerate((t0, t1, t2, t3)):
        d = jnp.abs(po[j] - tj)
        sl1 = sl1 + jnp.where(d < 1.0, 0.5 * d * d, d - 0.5)
    regp = jnp.sum(jnp.where(fg, sl1, 0.0))

    lg3 = lg_ref[0]
    acls = jnp.where(fg, lab.astype(jnp.int32), 0)
    cio = jax.lax.broadcasted_iota(jnp.int32, (C, L), 0)
    for i in range(RB):
        tilT = jnp.swapaxes(lg3[i], 0, 1)
        mxr = jnp.max(tilT, axis=0)
        er = jnp.exp(tilT - mxr[None, :])
        sr = jnp.sum(er, axis=0)
        lser = mxr + jnp.log(sr)
        ar = acls[i]
        pickr = jnp.sum(jnp.where(cio == ar[None, :], tilT, 0.0), axis=0)
        clsr = lser - pickr
        cls_ref[0, i] = clsr
        bits_ref[0, i] = jax.lax.bitcast_convert_type(
            jnp.maximum(clsr, 0.0), jnp.int32)

    @pl.when(first)
    def _():
        reg_ref[...] = jnp.zeros((1, 1), jnp.float32)
    reg_ref[...] += regp.reshape(1, 1)


def _mine_body(cls_hbm, bits_hbm, m_hbm, out_hbm, vals_v, m_v, bits_v,
               outv_v):
    wid = lax.axis_index("s") * 2 + lax.axis_index("c")
    A = vals_v.shape[0]
    NV = A // 16

    pltpu.sync_copy(cls_hbm.at[pl.ds(wid * A, A)], vals_v)
    pltpu.sync_copy(bits_hbm.at[pl.ds(wid * A, A)], bits_v)
    pltpu.sync_copy(m_hbm.at[pl.ds(wid * A, A)], m_v)

    zi16 = jnp.zeros((16,), jnp.int32)
    zf16 = jnp.zeros((16,), jnp.float32)
    iota = lax.broadcasted_iota(jnp.int32, (16,), 0)

    def ssum_i(vec):
        s = vec[0]
        for l in range(1, 16):
            s = s + vec[l]
        return s

    def ssum_f(vec):
        s = vec[0]
        for l in range(1, 16):
            s = s + vec[l]
        return s

    def smax_f(vec):
        s = vec[0]
        for l in range(1, 16):
            s = jnp.maximum(s, vec[l])
        return s

    def prep(i, acc):
        mm = m_v[pl.ds(i * 16, 16)]
        posm = mm >= 0
        b = jnp.where(posm, -1, bits_v[pl.ds(i * 16, 16)])
        bits_v[pl.ds(i * 16, 16)] = b
        return acc + jnp.where(posm, 1, 0)

    num_pos = ssum_i(lax.fori_loop(0, NV, prep, zi16))
    n_neg = A - num_pos
    k0 = jnp.minimum(_NEG_POS_RATIO * num_pos, n_neg)

    t = jnp.int32(0)
    for r in range(31):
        cand = t | jnp.int32(1 << (30 - r))

        def cbody(i, acc, cand=cand):
            for u in range(4):
                b = bits_v[pl.ds((i * 4 + u) * 16, 16)]
                acc = acc + jnp.where(b >= cand, 1, 0)
            return acc

        cnt = ssum_i(lax.fori_loop(0, NV // 4, cbody, zi16))
        t = jnp.where(cnt >= k0, cand, t)

    def fin(i, carry):
        psum, gsum, gcnt, tmax = carry
        for u in range(2):
            v = jnp.maximum(vals_v[pl.ds((i * 2 + u) * 16, 16)], 0.0)
            b = bits_v[pl.ds((i * 2 + u) * 16, 16)]
            psum = psum + jnp.where(b < 0, v, 0.0)
            gtm = b > t
            gsum = gsum + jnp.where(gtm, v, 0.0)
            gcnt = gcnt + jnp.where(gtm, 1, 0)
            tmax = jnp.maximum(tmax, jnp.where(b == t, v, 0.0))
        return (psum, gsum, gcnt, tmax)

    psum, gsum, gcnt, tmax = lax.fori_loop(0, NV // 2, fin,
                                           (zf16, zf16, zi16, zf16))
    pos_sum = ssum_f(psum)
    sum_gt = ssum_f(gsum)
    cnt_gt = ssum_i(gcnt)
    thr = smax_f(tmax)
    neg_sum = jnp.where(k0 > 0,
                        sum_gt + (k0 - cnt_gt).astype(jnp.float32) * thr,
                        0.0)
    outv_v[...] = jnp.where(
        iota == 0, pos_sum + neg_sum,
        jnp.where(iota == 1, num_pos.astype(jnp.float32), 0.0))
    pltpu.sync_copy(outv_v, out_hbm.at[pl.ds(wid * 16, 16)])


def _combine_body(rr_ref, reg_ref, out_ref):
    rr = rr_ref[...]
    s0 = jnp.sum(rr[:, 0:1])
    npos = jnp.sum(rr[:, 1:2])
    out_ref[...] = (_ALPHA * reg_ref[...]
                    + (s0 / jnp.maximum(npos, 1.0)).reshape(1, 1))


@jax.jit
def kernel(targets_bbox, targets_labels, pred_offsets, pred_cls_logits,
           anchors, matches):
    B, A, C = pred_cls_logits.shape
    G = targets_labels.shape[1]
    L = 128
    BLK = 8192
    RB = BLK // L

    tbl = jnp.concatenate(
        [jnp.transpose(targets_bbox, (0, 2, 1)),
         targets_labels.astype(jnp.float32)[:, None, :]], axis=1)
    lg4 = pred_cls_logits.reshape(B, A // L, L, C)
    anT = jnp.transpose(anchors, (0, 2, 1)).reshape(B, 4, A // L, L)
    poT = jnp.transpose(pred_offsets, (0, 2, 1)).reshape(B, 4, A // L, L)
    ml = matches.astype(jnp.int32).reshape(B, A // L, L)

    cls, clsbits, reg = pl.pallas_call(
        _phase1_body,
        grid=(B, A // BLK),
        in_specs=[
            pl.BlockSpec((1, 5, G), lambda b, j: (b, 0, 0),
                         memory_space=pltpu.SMEM),
            pl.BlockSpec((1, RB, L, C), lambda b, j: (b, j, 0, 0)),
            pl.BlockSpec((1, 4, RB, L), lambda b, j: (b, 0, j, 0)),
            pl.BlockSpec((1, 4, RB, L), lambda b, j: (b, 0, j, 0)),
            pl.BlockSpec((1, RB, L), lambda b, j: (b, j, 0)),
        ],
        out_specs=[
            pl.BlockSpec((1, RB, L), lambda b, j: (b, j, 0)),
            pl.BlockSpec((1, RB, L), lambda b, j: (b, j, 0)),
            pl.BlockSpec((1, 1), lambda b, j: (0, 0)),
        ],
        out_shape=[
            jax.ShapeDtypeStruct((B, A // L, L), jnp.float32),
            jax.ShapeDtypeStruct((B, A // L, L), jnp.int32),
            jax.ShapeDtypeStruct((1, 1), jnp.float32),
        ],
    )(tbl, lg4, anT, poT, ml)

    mine = functools.partial(
        pl.kernel,
        mesh=plsc.VectorSubcoreMesh(core_axis_name="c",
                                    subcore_axis_name="s"),
        out_type=jax.ShapeDtypeStruct((B * 16,), jnp.float32),
        scratch_types=[
            pltpu.VMEM((A,), jnp.float32),
            pltpu.VMEM((A,), jnp.int32),
            pltpu.VMEM((A,), jnp.int32),
            pltpu.VMEM((16,), jnp.float32),
        ],
    )(_mine_body)
    rowres = mine(cls.reshape(B * A), clsbits.reshape(B * A),
                  matches.astype(jnp.int32).reshape(B * A))

    out = pl.pallas_call(
        _combine_body,
        out_shape=jax.ShapeDtypeStruct((1, 1), jnp.float32),
    )(rowres.reshape(B, 16), reg)
    return out.reshape(())


# speedup vs baseline: 1.7753x; 1.0024x over previous
import functools

import jax
import jax.numpy as jnp
from jax import lax
from jax.experimental import pallas as pl
from jax.experimental.pallas import tpu as pltpu
from jax.experimental.pallas import tpu_sc as plsc

_NEG_POS_RATIO = 3
_ALPHA = 1.0


def _phase1_body(tbl_ref, lg_ref, anT_ref, poT_ref, ml_ref,
                 cls_ref, bits_ref, reg_ref):
    first = (pl.program_id(0) == 0) & (pl.program_id(1) == 0)
    RB = ml_ref.shape[1]
    L = ml_ref.shape[2]
    C = lg_ref.shape[3]
    G = tbl_ref.shape[2]

    m = ml_ref[0]
    fg = m >= 0
    safe = jnp.maximum(m, 0)

    zero = jnp.zeros((RB, L), jnp.float32)
    gx0 = zero
    gy0 = zero
    gx1 = zero
    gy1 = zero
    lab = zero
    for g in range(G):
        sel = safe == g
        gx0 = jnp.where(sel, tbl_ref[0, 0, g], gx0)
        gy0 = jnp.where(sel, tbl_ref[0, 1, g], gy0)
        gx1 = jnp.where(sel, tbl_ref[0, 2, g], gx1)
        gy1 = jnp.where(sel, tbl_ref[0, 3, g], gy1)
        lab = jnp.where(sel, tbl_ref[0, 4, g], lab)

    an = anT_ref[0]
    po = poT_ref[0]
    ax0 = an[0]
    ay0 = an[1]
    ax1 = an[2]
    ay1 = an[3]
    aw = ax1 - ax0
    ah = ay1 - ay0
    t0 = ((gx0 + gx1) - (ax0 + ax1)) * 0.5 / aw
    t1 = ((gy0 + gy1) - (ay0 + ay1)) * 0.5 / ah
    t2 = jnp.log((gx1 - gx0) / aw)
    t3 = jnp.log((gy1 - gy0) / ah)
    sl1 = zero
    for j, tj in enumerate((t0, t1, t2, t3)):
        d = jnp.abs(po[j] - tj)
        sl1 = sl1 + jnp.where(d < 1.0, 0.5 * d * d, d - 0.5)
    regp = jnp.sum(jnp.where(fg, sl1, 0.0))

    lg3 = lg_ref[0]
    acls = jnp.where(fg, lab.astype(jnp.int32), 0)
    cio = jax.lax.broadcasted_iota(jnp.int32, (C, L), 0)
    for i in range(RB):
        tilT = jnp.swapaxes(lg3[i], 0, 1)
        mxr = jnp.max(tilT, axis=0)
        er = jnp.exp(tilT - mxr[None, :])
        sr = jnp.sum(er, axis=0)
        lser = mxr + jnp.log(sr)
        ar = acls[i]
        pickr = jnp.sum(jnp.where(cio == ar[None, :], tilT, 0.0), axis=0)
        clsr = lser - pickr
        cls_ref[0, i] = clsr
        bits_ref[0, i] = jax.lax.bitcast_convert_type(
            jnp.maximum(clsr, 0.0), jnp.int32)

    @pl.when(first)
    def _():
        reg_ref[...] = jnp.zeros((1, 1), jnp.float32)
    reg_ref[...] += regp.reshape(1, 1)


def _mine_body(cls_hbm, bits_hbm, m_hbm, out_hbm, vals_v, m_v, bits_v,
               outv_v):
    wid = lax.axis_index("s") * 2 + lax.axis_index("c")
    A = vals_v.shape[0]
    NV = A // 16

    pltpu.sync_copy(cls_hbm.at[pl.ds(wid * A, A)], vals_v)
    pltpu.sync_copy(bits_hbm.at[pl.ds(wid * A, A)], bits_v)
    pltpu.sync_copy(m_hbm.at[pl.ds(wid * A, A)], m_v)

    zi16 = jnp.zeros((16,), jnp.int32)
    zf16 = jnp.zeros((16,), jnp.float32)
    iota = lax.broadcasted_iota(jnp.int32, (16,), 0)

    def ssum_i(vec):
        s = vec[0]
        for l in range(1, 16):
            s = s + vec[l]
        return s

    def ssum_f(vec):
        s = vec[0]
        for l in range(1, 16):
            s = s + vec[l]
        return s

    def smax_f(vec):
        s = vec[0]
        for l in range(1, 16):
            s = jnp.maximum(s, vec[l])
        return s

    def prep(i, acc):
        mm = m_v[pl.ds(i * 16, 16)]
        posm = mm >= 0
        b = jnp.where(posm, -1, bits_v[pl.ds(i * 16, 16)])
        bits_v[pl.ds(i * 16, 16)] = b
        return acc + jnp.where(posm, 1, 0)

    num_pos = ssum_i(lax.fori_loop(0, NV, prep, zi16))
    n_neg = A - num_pos
    k0 = jnp.minimum(_NEG_POS_RATIO * num_pos, n_neg)

    t = jnp.int32(0)
    for r in range(15):
        bhi = jnp.int32(1 << (30 - 2 * r))
        blo = jnp.int32(1 << (29 - 2 * r))
        c1 = t | blo
        c2 = t | bhi
        c3 = t | bhi | blo

        def cbody(i, accs, c1=c1, c2=c2, c3=c3):
            a1, a2, a3 = accs
            for u in range(4):
                b = bits_v[pl.ds((i * 4 + u) * 16, 16)]
                a1 = a1 + jnp.where(b >= c1, 1, 0)
                a2 = a2 + jnp.where(b >= c2, 1, 0)
                a3 = a3 + jnp.where(b >= c3, 1, 0)
            return (a1, a2, a3)

        a1, a2, a3 = lax.fori_loop(0, NV // 4, cbody, (zi16, zi16, zi16))
        n1 = ssum_i(a1)
        n2 = ssum_i(a2)
        n3 = ssum_i(a3)
        t = jnp.where(n3 >= k0, c3,
                      jnp.where(n2 >= k0, c2,
                                jnp.where(n1 >= k0, c1, t)))

    cand = t | jnp.int32(1)

    def cbody0(i, acc):
        for u in range(4):
            b = bits_v[pl.ds((i * 4 + u) * 16, 16)]
            acc = acc + jnp.where(b >= cand, 1, 0)
        return acc

    cnt = ssum_i(lax.fori_loop(0, NV // 4, cbody0, zi16))
    t = jnp.where(cnt >= k0, cand, t)

    def fin(i, carry):
        psum, gsum, gcnt, tmax = carry
        for u in range(2):
            v = jnp.maximum(vals_v[pl.ds((i * 2 + u) * 16, 16)], 0.0)
            b = bits_v[pl.ds((i * 2 + u) * 16, 16)]
            psum = psum + jnp.where(b < 0, v, 0.0)
            gtm = b > t
            gsum = gsum + jnp.where(gtm, v, 0.0)
            gcnt = gcnt + jnp.where(gtm, 1, 0)
            tmax = jnp.maximum(tmax, jnp.where(b == t, v, 0.0))
        return (psum, gsum, gcnt, tmax)

    psum, gsum, gcnt, tmax = lax.fori_loop(0, NV // 2, fin,
                                           (zf16, zf16, zi16, zf16))
    pos_sum = ssum_f(psum)
    sum_gt = ssum_f(gsum)
    cnt_gt = ssum_i(gcnt)
    thr = smax_f(tmax)
    neg_sum = jnp.where(k0 > 0,
                        sum_gt + (k0 - cnt_gt).astype(jnp.float32) * thr,
                        0.0)
    outv_v[...] = jnp.where(
        iota == 0, pos_sum + neg_sum,
        jnp.where(iota == 1, num_pos.astype(jnp.float32), 0.0))
    pltpu.sync_copy(outv_v, out_hbm.at[pl.ds(wid * 16, 16)])


def _combine_body(rr_ref, reg_ref, out_ref):
    rr = rr_ref[...]
    s0 = jnp.sum(rr[:, 0:1])
    npos = jnp.sum(rr[:, 1:2])
    out_ref[...] = (_ALPHA * reg_ref[...]
                    + (s0 / jnp.maximum(npos, 1.0)).reshape(1, 1))


@jax.jit
def kernel(targets_bbox, targets_labels, pred_offsets, pred_cls_logits,
           anchors, matches):
    B, A, C = pred_cls_logits.shape
    G = targets_labels.shape[1]
    L = 128
    BLK = 8192
    RB = BLK // L

    tbl = jnp.concatenate(
        [jnp.transpose(targets_bbox, (0, 2, 1)),
         targets_labels.astype(jnp.float32)[:, None, :]], axis=1)
    lg4 = pred_cls_logits.reshape(B, A // L, L, C)
    anT = jnp.transpose(anchors, (0, 2, 1)).reshape(B, 4, A // L, L)
    poT = jnp.transpose(pred_offsets, (0, 2, 1)).reshape(B, 4, A // L, L)
    ml = matches.astype(jnp.int32).reshape(B, A // L, L)

    cls, clsbits, reg = pl.pallas_call(
        _phase1_body,
        grid=(B, A // BLK),
        in_specs=[
            pl.BlockSpec((1, 5, G), lambda b, j: (b, 0, 0),
                         memory_space=pltpu.SMEM),
            pl.BlockSpec((1, RB, L, C), lambda b, j: (b, j, 0, 0)),
            pl.BlockSpec((1, 4, RB, L), lambda b, j: (b, 0, j, 0)),
            pl.BlockSpec((1, 4, RB, L), lambda b, j: (b, 0, j, 0)),
            pl.BlockSpec((1, RB, L), lambda b, j: (b, j, 0)),
        ],
        out_specs=[
            pl.BlockSpec((1, RB, L), lambda b, j: (b, j, 0)),
            pl.BlockSpec((1, RB, L), lambda b, j: (b, j, 0)),
            pl.BlockSpec((1, 1), lambda b, j: (0, 0)),
        ],
        out_shape=[
            jax.ShapeDtypeStruct((B, A // L, L), jnp.float32),
            jax.ShapeDtypeStruct((B, A // L, L), jnp.int32),
            jax.ShapeDtypeStruct((1, 1), jnp.float32),
        ],
    )(tbl, lg4, anT, poT, ml)

    mine = functools.partial(
        pl.kernel,
        mesh=plsc.VectorSubcoreMesh(core_axis_name="c",
                                    subcore_axis_name="s"),
        out_type=jax.ShapeDtypeStruct((B * 16,), jnp.float32),
        scratch_types=[
            pltpu.VMEM((A,), jnp.float32),
            pltpu.VMEM((A,), jnp.int32),
            pltpu.VMEM((A,), jnp.int32),
            pltpu.VMEM((16,), jnp.float32),
        ],
    )(_mine_body)
    rowres = mine(cls.reshape(B * A), clsbits.reshape(B * A),
                  matches.astype(jnp.int32).reshape(B * A))

    out = pl.pallas_call(
        _combine_body,
        out_shape=jax.ShapeDtypeStruct((1, 1), jnp.float32),
    )(rowres.reshape(B, 16), reg)
    return out.reshape(())


# speedup vs baseline: 1.8034x; 1.0158x over previous
import functools

import jax
import jax.numpy as jnp
from jax import lax
from jax.experimental import pallas as pl
from jax.experimental.pallas import tpu as pltpu
from jax.experimental.pallas import tpu_sc as plsc

_NEG_POS_RATIO = 3
_ALPHA = 1.0


def _phase1_body(tbl_ref, lg_ref, anT_ref, poT_ref, ml_ref,
                 cls_ref, bits_ref, reg_ref):
    first = (pl.program_id(0) == 0) & (pl.program_id(1) == 0)
    RB = ml_ref.shape[1]
    L = ml_ref.shape[2]
    C = lg_ref.shape[3]
    G = tbl_ref.shape[2]

    m = ml_ref[0]
    fg = m >= 0
    safe = jnp.maximum(m, 0)

    zero = jnp.zeros((RB, L), jnp.float32)
    gx0 = zero
    gy0 = zero
    gx1 = zero
    gy1 = zero
    lab = zero
    for g in range(G):
        sel = safe == g
        gx0 = jnp.where(sel, tbl_ref[0, 0, g], gx0)
        gy0 = jnp.where(sel, tbl_ref[0, 1, g], gy0)
        gx1 = jnp.where(sel, tbl_ref[0, 2, g], gx1)
        gy1 = jnp.where(sel, tbl_ref[0, 3, g], gy1)
        lab = jnp.where(sel, tbl_ref[0, 4, g], lab)

    an = anT_ref[0]
    po = poT_ref[0]
    ax0 = an[0]
    ay0 = an[1]
    ax1 = an[2]
    ay1 = an[3]
    aw = ax1 - ax0
    ah = ay1 - ay0
    t0 = ((gx0 + gx1) - (ax0 + ax1)) * 0.5 / aw
    t1 = ((gy0 + gy1) - (ay0 + ay1)) * 0.5 / ah
    t2 = jnp.log((gx1 - gx0) / aw)
    t3 = jnp.log((gy1 - gy0) / ah)
    sl1 = zero
    for j, tj in enumerate((t0, t1, t2, t3)):
        d = jnp.abs(po[j] - tj)
        sl1 = sl1 + jnp.where(d < 1.0, 0.5 * d * d, d - 0.5)
    regp = jnp.sum(jnp.where(fg, sl1, 0.0))

    lg3 = lg_ref[0]
    acls = jnp.where(fg, lab.astype(jnp.int32), 0)
    cio = jax.lax.broadcasted_iota(jnp.int32, (C, L), 0)
    for i in range(RB):
        tilT = jnp.swapaxes(lg3[i], 0, 1)
        mxr = jnp.max(tilT, axis=0)
        er = jnp.exp(tilT - mxr[None, :])
        sr = jnp.sum(er, axis=0)
        lser = mxr + jnp.log(sr)
        ar = acls[i]
        pickr = jnp.sum(jnp.where(cio == ar[None, :], tilT, 0.0), axis=0)
        clsr = lser - pickr
        cls_ref[0, i] = clsr
        bits_ref[0, i] = jnp.where(
            fg[i], -1,
            jax.lax.bitcast_convert_type(jnp.maximum(clsr, 0.0),
                                         jnp.int32))

    @pl.when(first)
    def _():
        reg_ref[...] = jnp.zeros((1, 1), jnp.float32)
    reg_ref[...] += regp.reshape(1, 1)


def _mine_body(cls_hbm, bits_hbm, out_hbm, vals_v, bits_v, outv_v):
    wid = lax.axis_index("s") * 2 + lax.axis_index("c")
    A = vals_v.shape[0]
    NV = A // 16

    pltpu.sync_copy(cls_hbm.at[pl.ds(wid * A, A)], vals_v)
    pltpu.sync_copy(bits_hbm.at[pl.ds(wid * A, A)], bits_v)

    zi16 = jnp.zeros((16,), jnp.int32)
    zf16 = jnp.zeros((16,), jnp.float32)
    iota = lax.broadcasted_iota(jnp.int32, (16,), 0)

    def ssum_i(vec):
        s = vec[0]
        for l in range(1, 16):
            s = s + vec[l]
        return s

    def ssum_f(vec):
        s = vec[0]
        for l in range(1, 16):
            s = s + vec[l]
        return s

    def smax_f(vec):
        s = vec[0]
        for l in range(1, 16):
            s = jnp.maximum(s, vec[l])
        return s

    def prep(i, acc):
        for u in range(4):
            b = bits_v[pl.ds((i * 4 + u) * 16, 16)]
            acc = acc + jnp.where(b < 0, 1, 0)
        return acc

    num_pos = ssum_i(lax.fori_loop(0, NV // 4, prep, zi16))
    n_neg = A - num_pos
    k0 = jnp.minimum(_NEG_POS_RATIO * num_pos, n_neg)

    t = jnp.int32(0)
    for r in range(15):
        bhi = jnp.int32(1 << (30 - 2 * r))
        blo = jnp.int32(1 << (29 - 2 * r))
        c1 = t | blo
        c2 = t | bhi
        c3 = t | bhi | blo

        def cbody(i, accs, c1=c1, c2=c2, c3=c3):
            a1, a2, a3 = accs
            for u in range(4):
                b = bits_v[pl.ds((i * 4 + u) * 16, 16)]
                a1 = a1 + jnp.where(b >= c1, 1, 0)
                a2 = a2 + jnp.where(b >= c2, 1, 0)
                a3 = a3 + jnp.where(b >= c3, 1, 0)
            return (a1, a2, a3)

        a1, a2, a3 = lax.fori_loop(0, NV // 4, cbody, (zi16, zi16, zi16))
        n1 = ssum_i(a1)
        n2 = ssum_i(a2)
        n3 = ssum_i(a3)
        t = jnp.where(n3 >= k0, c3,
                      jnp.where(n2 >= k0, c2,
                                jnp.where(n1 >= k0, c1, t)))

    cand = t | jnp.int32(1)

    def cbody0(i, acc):
        for u in range(4):
            b = bits_v[pl.ds((i * 4 + u) * 16, 16)]
            acc = acc + jnp.where(b >= cand, 1, 0)
        return acc

    cnt = ssum_i(lax.fori_loop(0, NV // 4, cbody0, zi16))
    t = jnp.where(cnt >= k0, cand, t)

    def fin(i, carry):
        psum, gsum, gcnt, tmax = carry
        for u in range(2):
            v = jnp.maximum(vals_v[pl.ds((i * 2 + u) * 16, 16)], 0.0)
            b = bits_v[pl.ds((i * 2 + u) * 16, 16)]
            psum = psum + jnp.where(b < 0, v, 0.0)
            gtm = b > t
            gsum = gsum + jnp.where(gtm, v, 0.0)
            gcnt = gcnt + jnp.where(gtm, 1, 0)
            tmax = jnp.maximum(tmax, jnp.where(b == t, v, 0.0))
        return (psum, gsum, gcnt, tmax)

    psum, gsum, gcnt, tmax = lax.fori_loop(0, NV // 2, fin,
                                           (zf16, zf16, zi16, zf16))
    pos_sum = ssum_f(psum)
    sum_gt = ssum_f(gsum)
    cnt_gt = ssum_i(gcnt)
    thr = smax_f(tmax)
    neg_sum = jnp.where(k0 > 0,
                        sum_gt + (k0 - cnt_gt).astype(jnp.float32) * thr,
                        0.0)
    outv_v[...] = jnp.where(
        iota == 0, pos_sum + neg_sum,
        jnp.where(iota == 1, num_pos.astype(jnp.float32), 0.0))
    pltpu.sync_copy(outv_v, out_hbm.at[pl.ds(wid * 16, 16)])


def _combine_body(rr_ref, reg_ref, out_ref):
    rr = rr_ref[...]
    s0 = jnp.sum(rr[:, 0:1])
    npos = jnp.sum(rr[:, 1:2])
    out_ref[...] = (_ALPHA * reg_ref[...]
                    + (s0 / jnp.maximum(npos, 1.0)).reshape(1, 1))


@jax.jit
def kernel(targets_bbox, targets_labels, pred_offsets, pred_cls_logits,
           anchors, matches):
    B, A, C = pred_cls_logits.shape
    G = targets_labels.shape[1]
    L = 128
    BLK = 8192
    RB = BLK // L

    tbl = jnp.concatenate(
        [jnp.transpose(targets_bbox, (0, 2, 1)),
         targets_labels.astype(jnp.float32)[:, None, :]], axis=1)
    lg4 = pred_cls_logits.reshape(B, A // L, L, C)
    anT = jnp.transpose(anchors, (0, 2, 1)).reshape(B, 4, A // L, L)
    poT = jnp.transpose(pred_offsets, (0, 2, 1)).reshape(B, 4, A // L, L)
    ml = matches.astype(jnp.int32).reshape(B, A // L, L)

    cls, clsbits, reg = pl.pallas_call(
        _phase1_body,
        grid=(B, A // BLK),
        in_specs=[
            pl.BlockSpec((1, 5, G), lambda b, j: (b, 0, 0),
                         memory_space=pltpu.SMEM),
            pl.BlockSpec((1, RB, L, C), lambda b, j: (b, j, 0, 0)),
            pl.BlockSpec((1, 4, RB, L), lambda b, j: (b, 0, j, 0)),
            pl.BlockSpec((1, 4, RB, L), lambda b, j: (b, 0, j, 0)),
            pl.BlockSpec((1, RB, L), lambda b, j: (b, j, 0)),
        ],
        out_specs=[
            pl.BlockSpec((1, RB, L), lambda b, j: (b, j, 0)),
            pl.BlockSpec((1, RB, L), lambda b, j: (b, j, 0)),
            pl.BlockSpec((1, 1), lambda b, j: (0, 0)),
        ],
        out_shape=[
            jax.ShapeDtypeStruct((B, A // L, L), jnp.float32),
            jax.ShapeDtypeStruct((B, A // L, L), jnp.int32),
            jax.ShapeDtypeStruct((1, 1), jnp.float32),
        ],
    )(tbl, lg4, anT, poT, ml)

    mine = functools.partial(
        pl.kernel,
        mesh=plsc.VectorSubcoreMesh(core_axis_name="c",
                                    subcore_axis_name="s"),
        out_type=jax.ShapeDtypeStruct((B * 16,), jnp.float32),
        scratch_types=[
            pltpu.VMEM((A,), jnp.float32),
            pltpu.VMEM((A,), jnp.int32),
            pltpu.VMEM((16,), jnp.float32),
        ],
    )(_mine_body)
    rowres = mine(cls.reshape(B * A), clsbits.reshape(B * A))

    out = pl.pallas_call(
        _combine_body,
        out_shape=jax.ShapeDtypeStruct((1, 1), jnp.float32),
    )(rowres.reshape(B, 16), reg)
    return out.reshape(())
